# baseline (speedup 1.0000x reference)
import functools

import jax
import jax.numpy as jnp
from jax.experimental import pallas as pl
from jax.experimental.pallas import tpu as pltpu
from jax.experimental.pallas import tpu_sc as plsc

def _dot(a, b):
    return jnp.dot(a, b, preferred_element_type=jnp.float32)


_bdot = _dot


def _dot_t(a, b):
    return jax.lax.dot_general(a, b, (((1,), (1,)), ((), ())),
                               preferred_element_type=jnp.float32)


_BPC = 2
_NPB = 2048


def _dot0(a, b):
    return jax.lax.dot_general(a, b, (((0,), (0,)), ((), ())),
                               preferred_element_type=jnp.float32)


def _obj_pn_kernel(x_ref, wt1, bt1, wt2, bt2, wt3, bt3,
                   wp1, bp1, wp2, bp2, wp3, bp3, cb6_ref, ot_ref, op_ref,
                   d6_ref):
    def chain(xb, w1, b1, w2, b2, w3, b3):
        h = jnp.maximum(_dot0(xb, w1[...]) + b1[...], 0.0)
        h = jnp.maximum(_bdot(h, w2[...]) + b2[...], 0.0)
        h = _bdot(h, w3[...]) + b3[...]
        return jnp.max(h, axis=0, keepdims=True)

    pts, pps = [], []
    for i in range(_BPC):
        xb = x_ref[i]
        pts.append(chain(xb, wt1, bt1, wt2, bt2, wt3, bt3))
        pps.append(chain(xb, wp1, bp1, wp2, bp2, wp3, bp3))
    ot_ref[...] = jnp.stack(pts)
    opos = jnp.concatenate(pps, axis=0)
    op_ref[...] = opos[:, None, :]
    cb6 = cb6_ref[...]
    d6 = (jnp.sum(opos * opos, axis=1, keepdims=True)
          - 2.0 * _dot_t(opos, cb6)
          + jnp.sum(cb6 * cb6, axis=1)[None, :])
    lane = jax.lax.broadcasted_iota(jnp.int32, d6.shape, 1)
    dmin = jnp.min(d6, axis=1, keepdims=True)
    idx = jnp.min(jnp.where(d6 == dmin, lane, 128), axis=1)
    d6_ref[...] = idx.reshape(_BPC, 1, 1)


def _obj_pointnets(obj_pc, pt_t, pt_p, cb6):
    B, C, N = obj_pc.shape
    steps = B // _BPC
    full = lambda s: pl.BlockSpec(s, lambda c: (0,) * len(s))
    wspecs = []
    args = []
    for p in (pt_t, pt_p):
        for k in ('w1', 'b1', 'w2', 'b2', 'w3', 'b3'):
            a = p[k]
            if a.ndim == 1:
                a = a.reshape(1, -1)
            args.append(a)
            wspecs.append(full(a.shape))
    ot, op, d6 = pl.pallas_call(
        _obj_pn_kernel,
        grid=(steps,),
        in_specs=[pl.BlockSpec((_BPC, C, N), lambda c: (c, 0, 0))] + wspecs
                 + [full(cb6.shape)],
        out_specs=[pl.BlockSpec((_BPC, 1, 1024), lambda c: (c, 0, 0)),
                   pl.BlockSpec((_BPC, 1, 1024), lambda c: (c, 0, 0)),
                   pl.BlockSpec((_BPC, 1, 1), lambda c: (c, 0, 0))],
        out_shape=[jax.ShapeDtypeStruct((B, 1, 1024), jnp.float32),
                   jax.ShapeDtypeStruct((B, 1, 1024), jnp.float32),
                   jax.ShapeDtypeStruct((B, 1, 1), jnp.int32)],
    )(obj_pc, *args, cb6)
    return ot.reshape(B, 1024), op.reshape(B, 1024), d6.reshape(B)


_FJ = [0, 1, 2, 3, 4, 5, 5]
_STARTS = [0, 83, 206, 326, 448, 569, 697]
_VALID = [83, 123, 120, 122, 121, 128, 81]


def _hand_pn_kernel(chunk_ref, nat_ref, w1, b1, w2, b2, w3, b3, out_ref):
    j = pl.program_id(0)
    v = jnp.int32(_VALID[-1])
    for jj in range(6):
        v = jnp.where(j == jj, jnp.int32(_VALID[jj]), v)
    nat = nat_ref[...]
    mean = jnp.sum(nat, axis=2) * (1.0 / 778.0)
    B = nat.shape[0]
    x = chunk_ref[0].reshape(B, 128, 3) - mean[:, None, :]
    x = x.reshape(B * 128, 3)
    h = jnp.maximum(_bdot(x, w1[0]) + b1[0], 0.0)
    h = jnp.maximum(_bdot(h, w2[0]) + b2[0], 0.0)
    h = _bdot(h, w3[0]) + b3[0]
    h = h.reshape(B, 128, 1024)
    pid = jax.lax.broadcasted_iota(jnp.int32, (B, 128, 1), 1)
    h = jnp.where(pid < v, h, -1e30)
    pm = jnp.max(h, axis=1)

    @pl.when(j < 6)
    def _():
        out_ref[0] = pm

    @pl.when(j == 6)
    def _():
        out_ref[0] = jnp.maximum(out_ref[0], pm)


def _hand_pointnets(hand_xyz, enc):
    B = hand_xyz.shape[0]
    hp = jnp.transpose(hand_xyz, (0, 2, 1))
    hp = jnp.pad(hp, ((0, 0), (0, 832 - 778), (0, 0)))
    chunks = jnp.stack([hp[:, s:s + 128, :] for s in _STARTS])
    chunks = chunks.reshape(7, B * 128, 3)
    nat = jnp.pad(hand_xyz, ((0, 0), (0, 0), (0, 832 - 778)))

    stk = lambda k: jnp.stack([enc[i][k] for i in range(6)])
    W1, W2, W3 = stk('w1'), stk('w2'), stk('w3')
    B1 = stk('b1')[:, None, :]
    B2 = stk('b2')[:, None, :]
    B3 = stk('b3')[:, None, :]

    wmap = lambda j: (jnp.minimum(j, 5), 0, 0)
    return pl.pallas_call(
        _hand_pn_kernel,
        grid=(7,),
        in_specs=[
            pl.BlockSpec((1, B * 128, 3), lambda j: (j, 0, 0)),
            pl.BlockSpec(nat.shape, lambda j: (0, 0, 0)),
            pl.BlockSpec((1,) + W1.shape[1:], wmap),
            pl.BlockSpec((1,) + B1.shape[1:], wmap),
            pl.BlockSpec((1,) + W2.shape[1:], wmap),
            pl.BlockSpec((1,) + B2.shape[1:], wmap),
            pl.BlockSpec((1,) + W3.shape[1:], wmap),
            pl.BlockSpec((1,) + B3.shape[1:], wmap),
        ],
        out_specs=pl.BlockSpec((1, B, 1024), wmap),
        out_shape=jax.ShapeDtypeStruct((6, B, 1024), jnp.float32),
    )(chunks, nat, W1, B1, W2, B2, W3, B3)


def _emb_vq_kernel(feat_ref, w0, b0, wm, bm, cb_ref, z_ref, d_ref):
    f = feat_ref[0]
    h = jnp.maximum(_bdot(f, w0[0]) + b0[0], 0.0)
    z = _bdot(h, wm[0]) + bm[0]
    cb = cb_ref[0]
    d = (jnp.sum(z * z, axis=1, keepdims=True)
         - 2.0 * _dot_t(z, cb)
         + jnp.sum(cb * cb, axis=1)[None, :])
    B = d.shape[0]
    lane = jax.lax.broadcasted_iota(jnp.int32, (B, 128), 1)
    dmin = jnp.min(d, axis=1, keepdims=True)
    idx = jnp.min(jnp.where(d == dmin, lane, 128), axis=1)
    z_ref[0] = z
    d_ref[0] = (idx + 128 * pl.program_id(0)).reshape(1, B)


def _emb_vq(feat, emb, cbs):
    B = feat.shape[1]
    stk = lambda k: jnp.stack([emb[i][k] for i in range(6)])
    W0, WM = stk('w0'), stk('wm')
    B0 = stk('b0')[:, None, :]
    BM = stk('bm')[:, None, :]
    CB = jnp.stack(cbs)
    bmap = lambda i: (i, 0, 0)
    return pl.pallas_call(
        _emb_vq_kernel,
        grid=(6,),
        in_specs=[
            pl.BlockSpec((1, B, 1024), bmap),
            pl.BlockSpec((1,) + W0.shape[1:], bmap),
            pl.BlockSpec((1,) + B0.shape[1:], bmap),
            pl.BlockSpec((1,) + WM.shape[1:], bmap),
            pl.BlockSpec((1,) + BM.shape[1:], bmap),
            pl.BlockSpec((1,) + CB.shape[1:], bmap),
        ],
        out_specs=[pl.BlockSpec((1, B, 256), bmap),
                   pl.BlockSpec((1, 1, B), bmap)],
        out_shape=[jax.ShapeDtypeStruct((6, B, 256), jnp.float32),
                   jax.ShapeDtypeStruct((6, 1, B), jnp.int32)],
    )(feat, W0, B0, WM, BM, CB)


def _sc_vq(idxFp, idx6, cbF, cb6):
    mesh = plsc.VectorSubcoreMesh(core_axis_name="c", subcore_axis_name="s")

    @functools.partial(
        pl.kernel, mesh=mesh,
        out_type=[jax.ShapeDtypeStruct((256, 256), jnp.float32),
                  jax.ShapeDtypeStruct((32, 1024), jnp.float32)],
        scratch_types=[pltpu.VMEM((8,), jnp.int32),
                       pltpu.VMEM((8, 256), jnp.float32),
                       pltpu.VMEM((8,), jnp.int32),
                       pltpu.VMEM((8, 1024), jnp.float32),
                       pltpu.SemaphoreType.DMA],
    )
    def k(idxF_hbm, idx6_hbm, cbF_hbm, cb6_hbm, qFp_hbm, q6_hbm,
          idxv, rows, idxv6, rows6, sem):
        wid = jax.lax.axis_index("s") * 2 + jax.lax.axis_index("c")
        pltpu.sync_copy(idxF_hbm.at[pl.ds(wid * 8, 8)], idxv)
        pltpu.async_copy(cbF_hbm.at[idxv], rows, sem).wait()
        pltpu.sync_copy(rows, qFp_hbm.at[pl.ds(wid * 8, 8)])

        @pl.when(wid < 4)
        def _():
            pltpu.sync_copy(idx6_hbm.at[pl.ds(wid * 8, 8)], idxv6)
            pltpu.async_copy(cb6_hbm.at[idxv6], rows6, sem).wait()
            pltpu.sync_copy(rows6, q6_hbm.at[pl.ds(wid * 8, 8)])

    return k(idxFp, idx6, cbF, cb6)


def _final_kernel(q_ref, z_ref, q6_ref, ot_ref, op_ref,
                  d0, db0, d1, db1, d2p, db2p,
                  p0, pb0, p1, pb1, p2p, pb2p, out_ref, loss_ref):
    opos = op_ref[...]
    q6 = q6_ref[...]
    B = q6.shape[0]
    ssq = jnp.sum((q_ref[...] - z_ref[...]) ** 2)
    ssq6 = jnp.sum((q6 - opos) ** 2)
    loss = (1.25 * ssq / (B * 256.0)
            + 3.0 * ssq6 / (B * 1024.0))

    otype = ot_ref[...]
    x1 = _bdot(otype, d0[1536:, :]) + db0[...]
    for i in range(6):
        x1 = x1 + _bdot(q_ref[i], d0[256 * i:256 * (i + 1), :])
    h = jnp.maximum(x1, 0.0)
    h = jnp.maximum(_bdot(h, d1[...]) + db1[...], 0.0)
    recon = _bdot(h, d2p[...]) + db2p[...]
    y1 = _bdot(q6, p0[:1024, :]) + _bdot(otype, p0[1024:, :]) + pb0[...]
    g = jnp.maximum(y1, 0.0)
    g = jnp.maximum(_bdot(g, p1[...]) + pb1[...], 0.0)
    pos = _bdot(g, p2p[...]) + pb2p[...]
    out_ref[...] = recon + pos
    loss_ref[...] = loss.reshape(1, 1)


def _final(q, z, q6, otype, opos, dec, pos_dec):
    B = otype.shape[0]
    d2p = jnp.pad(dec['w2'], ((0, 0), (0, 6)))
    db2p = jnp.pad(dec['b2'], (0, 6)).reshape(1, 61)
    p2p = jnp.pad(pos_dec['w2'], ((0, 0), (55, 0)))
    pb2p = jnp.pad(pos_dec['b2'], (55, 0)).reshape(1, 61)
    args = [q, z, q6, otype, opos,
            dec['w0'], dec['b0'].reshape(1, -1),
            dec['w1'], dec['b1'].reshape(1, -1), d2p, db2p,
            pos_dec['w0'], pos_dec['b0'].reshape(1, -1),
            pos_dec['w1'], pos_dec['b1'].reshape(1, -1), p2p, pb2p]
    out, loss = pl.pallas_call(
        _final_kernel,
        in_specs=[pl.BlockSpec(a.shape, functools.partial(lambda n: (0,) * n, a.ndim))
                  for a in args],
        out_specs=[pl.BlockSpec((B, 61), lambda: (0, 0)),
                   pl.BlockSpec((1, 1), lambda: (0, 0))],
        out_shape=[jax.ShapeDtypeStruct((B, 61), jnp.float32),
                   jax.ShapeDtypeStruct((1, 1), jnp.float32)],
    )(*args)
    return out, loss


def kernel(obj_pc, hand_xyz, params):
    otype, opos, idx6 = _obj_pointnets(obj_pc, params['obj_type'],
                                       params['obj_pos'], params['cb6'])
    feat = _hand_pointnets(hand_xyz, params['hand_enc'])
    z, idxF = _emb_vq(feat, params['emb'], params['cb'])
    cbF = jnp.concatenate(params['cb'], axis=0)
    idxFp = jnp.pad(idxF.reshape(192).reshape(32, 6), ((0, 0), (0, 2)))
    qFp, q6 = _sc_vq(idxFp.reshape(256), idx6, cbF, params['cb6'])
    qF = qFp.reshape(32, 8, 256)[:, :6, :].reshape(6, 32, 256)
    out, loss = _final(qF, z, q6, otype, opos,
                       params['dec'], params['pos_dec'])
    return out, loss[0, 0]


# speedup vs baseline: 1.0161x; 1.0161x over previous
import functools

import jax
import jax.numpy as jnp
from jax.experimental import pallas as pl
from jax.experimental.pallas import tpu as pltpu
from jax.experimental.pallas import tpu_sc as plsc

def _dot(a, b):
    return jnp.dot(a, b, preferred_element_type=jnp.float32)


_bdot = _dot


def _dot_t(a, b):
    return jax.lax.dot_general(a, b, (((1,), (1,)), ((), ())),
                               preferred_element_type=jnp.float32)


_BPC = 2
_NPB = 2048


def _dot0(a, b):
    return jax.lax.dot_general(a, b, (((0,), (0,)), ((), ())),
                               preferred_element_type=jnp.float32)


def _obj_pn_kernel(x_ref, wt1, bt1, wt2, bt2, wt3, bt3,
                   wp1, bp1, wp2, bp2, wp3, bp3, cb6_ref, ot_ref, op_ref,
                   d6_ref):
    def chain(xb, w1, b1, w2, b2, w3, b3):
        h = jnp.maximum(_dot0(xb, w1[...]) + b1[...], 0.0)
        h = jnp.maximum(_bdot(h, w2[...]) + b2[...], 0.0)
        h = _bdot(h, w3[...]) + b3[...]
        return jnp.max(h, axis=0, keepdims=True)

    pts, pps = [], []
    for i in range(_BPC):
        xb = x_ref[i]
        pts.append(chain(xb, wt1, bt1, wt2, bt2, wt3, bt3))
        pps.append(chain(xb, wp1, bp1, wp2, bp2, wp3, bp3))
    ot_ref[...] = jnp.stack(pts)
    opos = jnp.concatenate(pps, axis=0)
    op_ref[...] = opos[:, None, :]
    cb6 = cb6_ref[...]
    d6 = (jnp.sum(opos * opos, axis=1, keepdims=True)
          - 2.0 * _dot_t(opos, cb6)
          + jnp.sum(cb6 * cb6, axis=1)[None, :])
    lane = jax.lax.broadcasted_iota(jnp.int32, d6.shape, 1)
    dmin = jnp.min(d6, axis=1, keepdims=True)
    idx = jnp.min(jnp.where(d6 == dmin, lane, 128), axis=1)
    d6_ref[...] = idx.reshape(_BPC, 1, 1)


def _obj_pointnets(obj_pc, pt_t, pt_p, cb6):
    B, C, N = obj_pc.shape
    steps = B // _BPC
    full = lambda s: pl.BlockSpec(s, lambda c: (0,) * len(s))
    wspecs = []
    args = []
    for p in (pt_t, pt_p):
        for k in ('w1', 'b1', 'w2', 'b2', 'w3', 'b3'):
            a = p[k]
            if a.ndim == 1:
                a = a.reshape(1, -1)
            args.append(a)
            wspecs.append(full(a.shape))
    ot, op, d6 = pl.pallas_call(
        _obj_pn_kernel,
        grid=(steps,),
        in_specs=[pl.BlockSpec((_BPC, C, N), lambda c: (c, 0, 0))] + wspecs
                 + [full(cb6.shape)],
        out_specs=[pl.BlockSpec((_BPC, 1, 1024), lambda c: (c, 0, 0)),
                   pl.BlockSpec((_BPC, 1, 1024), lambda c: (c, 0, 0)),
                   pl.BlockSpec((_BPC, 1, 1), lambda c: (c, 0, 0))],
        out_shape=[jax.ShapeDtypeStruct((B, 1, 1024), jnp.float32),
                   jax.ShapeDtypeStruct((B, 1, 1024), jnp.float32),
                   jax.ShapeDtypeStruct((B, 1, 1), jnp.int32)],
    )(obj_pc, *args, cb6)
    return ot.reshape(B, 1024), op.reshape(B, 1024), d6.reshape(B)


_FJ = [0, 1, 2, 3, 4, 5, 5]
_STARTS = [0, 83, 206, 326, 448, 569, 697]
_VALID = [83, 123, 120, 122, 121, 128, 81]


def _hand_pn_kernel(chunk_ref, nat_ref, w1, b1, w2, b2, w3, b3, out_ref):
    j = pl.program_id(0)
    v = jnp.int32(_VALID[-1])
    for jj in range(6):
        v = jnp.where(j == jj, jnp.int32(_VALID[jj]), v)
    nat = nat_ref[...]
    mean = jnp.sum(nat, axis=2) * (1.0 / 778.0)
    B = nat.shape[0]
    x = chunk_ref[0].reshape(B, 128, 3) - mean[:, None, :]
    x = x.reshape(B * 128, 3)
    h = jnp.maximum(_bdot(x, w1[0]) + b1[0], 0.0)
    h = jnp.maximum(_bdot(h, w2[0]) + b2[0], 0.0)
    h = _bdot(h, w3[0]) + b3[0]
    h = h.reshape(B, 128, 1024)
    pid = jax.lax.broadcasted_iota(jnp.int32, (B, 128, 1), 1)
    h = jnp.where(pid < v, h, -1e30)
    pm = jnp.max(h, axis=1)

    @pl.when(j < 6)
    def _():
        out_ref[0] = pm

    @pl.when(j == 6)
    def _():
        out_ref[0] = jnp.maximum(out_ref[0], pm)


def _hand_pointnets(hand_xyz, enc):
    B = hand_xyz.shape[0]
    hp = jnp.transpose(hand_xyz, (0, 2, 1))
    hp = jnp.pad(hp, ((0, 0), (0, 832 - 778), (0, 0)))
    chunks = jnp.stack([hp[:, s:s + 128, :] for s in _STARTS])
    chunks = chunks.reshape(7, B * 128, 3)
    nat = jnp.pad(hand_xyz, ((0, 0), (0, 0), (0, 832 - 778)))

    stk = lambda k: jnp.stack([enc[i][k] for i in range(6)])
    W1, W2, W3 = stk('w1'), stk('w2'), stk('w3')
    B1 = stk('b1')[:, None, :]
    B2 = stk('b2')[:, None, :]
    B3 = stk('b3')[:, None, :]

    wmap = lambda j: (jnp.minimum(j, 5), 0, 0)
    return pl.pallas_call(
        _hand_pn_kernel,
        grid=(7,),
        in_specs=[
            pl.BlockSpec((1, B * 128, 3), lambda j: (j, 0, 0)),
            pl.BlockSpec(nat.shape, lambda j: (0, 0, 0)),
            pl.BlockSpec((1,) + W1.shape[1:], wmap),
            pl.BlockSpec((1,) + B1.shape[1:], wmap),
            pl.BlockSpec((1,) + W2.shape[1:], wmap),
            pl.BlockSpec((1,) + B2.shape[1:], wmap),
            pl.BlockSpec((1,) + W3.shape[1:], wmap),
            pl.BlockSpec((1,) + B3.shape[1:], wmap),
        ],
        out_specs=pl.BlockSpec((1, B, 1024), wmap),
        out_shape=jax.ShapeDtypeStruct((6, B, 1024), jnp.float32),
    )(chunks, nat, W1, B1, W2, B2, W3, B3)


def _emb_vq_kernel(feat_ref, w0, b0, wm, bm, cb_ref, z_ref, d_ref):
    f = feat_ref[0]
    h = jnp.maximum(_bdot(f, w0[0]) + b0[0], 0.0)
    z = _bdot(h, wm[0]) + bm[0]
    cb = cb_ref[0]
    d = (jnp.sum(z * z, axis=1, keepdims=True)
         - 2.0 * _dot_t(z, cb)
         + jnp.sum(cb * cb, axis=1)[None, :])
    B = d.shape[0]
    lane = jax.lax.broadcasted_iota(jnp.int32, (B, 128), 1)
    dmin = jnp.min(d, axis=1, keepdims=True)
    idx = jnp.min(jnp.where(d == dmin, lane, 128), axis=1)
    z_ref[0] = z
    d_ref[0] = (idx + 128 * pl.program_id(0)).reshape(1, B)


def _emb_vq(feat, emb, cbs):
    B = feat.shape[1]
    stk = lambda k: jnp.stack([emb[i][k] for i in range(6)])
    W0, WM = stk('w0'), stk('wm')
    B0 = stk('b0')[:, None, :]
    BM = stk('bm')[:, None, :]
    CB = jnp.stack(cbs)
    bmap = lambda i: (i, 0, 0)
    return pl.pallas_call(
        _emb_vq_kernel,
        grid=(6,),
        in_specs=[
            pl.BlockSpec((1, B, 1024), bmap),
            pl.BlockSpec((1,) + W0.shape[1:], bmap),
            pl.BlockSpec((1,) + B0.shape[1:], bmap),
            pl.BlockSpec((1,) + WM.shape[1:], bmap),
            pl.BlockSpec((1,) + BM.shape[1:], bmap),
            pl.BlockSpec((1,) + CB.shape[1:], bmap),
        ],
        out_specs=[pl.BlockSpec((1, B, 256), bmap),
                   pl.BlockSpec((1, 1, B), bmap)],
        out_shape=[jax.ShapeDtypeStruct((6, B, 256), jnp.float32),
                   jax.ShapeDtypeStruct((6, 1, B), jnp.int32)],
    )(feat, W0, B0, WM, BM, CB)


def _sc_vq(idxFp, cbF):
    mesh = plsc.VectorSubcoreMesh(core_axis_name="c", subcore_axis_name="s")

    @functools.partial(
        pl.kernel, mesh=mesh,
        out_type=jax.ShapeDtypeStruct((256, 256), jnp.float32),
        scratch_types=[pltpu.VMEM((8,), jnp.int32),
                       pltpu.VMEM((8, 256), jnp.float32),
                       pltpu.SemaphoreType.DMA],
    )
    def k(idxF_hbm, cbF_hbm, qFp_hbm, idxv, rows, sem):
        wid = jax.lax.axis_index("s") * 2 + jax.lax.axis_index("c")
        pltpu.sync_copy(idxF_hbm.at[pl.ds(wid * 8, 8)], idxv)
        pltpu.async_copy(cbF_hbm.at[idxv], rows, sem).wait()
        pltpu.sync_copy(rows, qFp_hbm.at[pl.ds(wid * 8, 8)])

    return k(idxFp, cbF)


def _final_kernel(q_ref, z_ref, idx6_ref, cb6_ref, ot_ref, op_ref,
                  d0, db0, d1, db1, d2p, db2p,
                  p0, pb0, p1, pb1, p2p, pb2p, out_ref, loss_ref):
    opos = op_ref[...]
    B = opos.shape[0]
    lane = jax.lax.broadcasted_iota(jnp.int32, (B, 128), 1)
    onehot = (lane == idx6_ref[...]).astype(jnp.float32)
    q6 = _dot(onehot, cb6_ref[...])
    ssq = jnp.sum((q_ref[...] - z_ref[...]) ** 2)
    ssq6 = jnp.sum((q6 - opos) ** 2)
    loss = (1.25 * ssq / (B * 256.0)
            + 3.0 * ssq6 / (B * 1024.0))

    otype = ot_ref[...]
    x1 = _bdot(otype, d0[1536:, :]) + db0[...]
    for i in range(6):
        x1 = x1 + _bdot(q_ref[i], d0[256 * i:256 * (i + 1), :])
    h = jnp.maximum(x1, 0.0)
    h = jnp.maximum(_bdot(h, d1[...]) + db1[...], 0.0)
    recon = _bdot(h, d2p[...]) + db2p[...]
    y1 = _bdot(q6, p0[:1024, :]) + _bdot(otype, p0[1024:, :]) + pb0[...]
    g = jnp.maximum(y1, 0.0)
    g = jnp.maximum(_bdot(g, p1[...]) + pb1[...], 0.0)
    pos = _bdot(g, p2p[...]) + pb2p[...]
    out_ref[...] = recon + pos
    loss_ref[...] = loss.reshape(1, 1)


def _final(q, z, idx6, cb6, otype, opos, dec, pos_dec):
    B = otype.shape[0]
    d2p = jnp.pad(dec['w2'], ((0, 0), (0, 6)))
    db2p = jnp.pad(dec['b2'], (0, 6)).reshape(1, 61)
    p2p = jnp.pad(pos_dec['w2'], ((0, 0), (55, 0)))
    pb2p = jnp.pad(pos_dec['b2'], (55, 0)).reshape(1, 61)
    args = [q, z, idx6, cb6, otype, opos,
            dec['w0'], dec['b0'].reshape(1, -1),
            dec['w1'], dec['b1'].reshape(1, -1), d2p, db2p,
            pos_dec['w0'], pos_dec['b0'].reshape(1, -1),
            pos_dec['w1'], pos_dec['b1'].reshape(1, -1), p2p, pb2p]
    out, loss = pl.pallas_call(
        _final_kernel,
        in_specs=[pl.BlockSpec(a.shape, functools.partial(lambda n: (0,) * n, a.ndim))
                  for a in args],
        out_specs=[pl.BlockSpec((B, 61), lambda: (0, 0)),
                   pl.BlockSpec((1, 1), lambda: (0, 0))],
        out_shape=[jax.ShapeDtypeStruct((B, 61), jnp.float32),
                   jax.ShapeDtypeStruct((1, 1), jnp.float32)],
    )(*args)
    return out, loss


def kernel(obj_pc, hand_xyz, params):
    otype, opos, idx6 = _obj_pointnets(obj_pc, params['obj_type'],
                                       params['obj_pos'], params['cb6'])
    feat = _hand_pointnets(hand_xyz, params['hand_enc'])
    z, idxF = _emb_vq(feat, params['emb'], params['cb'])
    cbF = jnp.concatenate(params['cb'], axis=0)
    idxFp = jnp.pad(idxF.reshape(192).reshape(32, 6), ((0, 0), (0, 2)))
    qFp = _sc_vq(idxFp.reshape(256), cbF)
    qF = qFp.reshape(32, 8, 256)[:, :6, :].reshape(6, 32, 256)
    out, loss = _final(qF, z, idx6.reshape(32, 1), params['cb6'], otype, opos,
                       params['dec'], params['pos_dec'])
    return out, loss[0, 0]


# speedup vs baseline: 1.0238x; 1.0076x over previous
import functools

import jax
import jax.numpy as jnp
from jax.experimental import pallas as pl
from jax.experimental.pallas import tpu as pltpu
from jax.experimental.pallas import tpu_sc as plsc

def _dot(a, b):
    return jnp.dot(a, b, preferred_element_type=jnp.float32)


_bdot = _dot


def _dot_t(a, b):
    return jax.lax.dot_general(a, b, (((1,), (1,)), ((), ())),
                               preferred_element_type=jnp.float32)


_BPC = 2
_NPB = 2048


def _dot0(a, b):
    return jax.lax.dot_general(a, b, (((0,), (0,)), ((), ())),
                               preferred_element_type=jnp.float32)


def _obj_pn_kernel(x_ref, wt1, bt1, wt2, bt2, wt3, bt3,
                   wp1, bp1, wp2, bp2, wp3, bp3, cb6_ref, ot_ref, op_ref,
                   d6_ref):
    def chain(xb, w1, b1, w2, b2, w3, b3):
        h = jnp.maximum(_dot0(xb, w1[...]) + b1[...], 0.0)
        h = jnp.maximum(_bdot(h, w2[...]) + b2[...], 0.0)
        h = _bdot(h, w3[...]) + b3[...]
        return jnp.max(h, axis=0, keepdims=True)

    pts, pps = [], []
    for i in range(_BPC):
        xb = x_ref[i]
        pts.append(chain(xb, wt1, bt1, wt2, bt2, wt3, bt3))
        pps.append(chain(xb, wp1, bp1, wp2, bp2, wp3, bp3))
    ot_ref[...] = jnp.stack(pts)
    opos = jnp.concatenate(pps, axis=0)
    op_ref[...] = opos[:, None, :]
    cb6 = cb6_ref[...]
    d6 = (jnp.sum(opos * opos, axis=1, keepdims=True)
          - 2.0 * _dot_t(opos, cb6)
          + jnp.sum(cb6 * cb6, axis=1)[None, :])
    lane = jax.lax.broadcasted_iota(jnp.int32, d6.shape, 1)
    dmin = jnp.min(d6, axis=1, keepdims=True)
    idx = jnp.min(jnp.where(d6 == dmin, lane, 128), axis=1)
    d6_ref[...] = idx.reshape(_BPC, 1, 1)


def _obj_pointnets(obj_pc, pt_t, pt_p, cb6):
    B, C, N = obj_pc.shape
    steps = B // _BPC
    full = lambda s: pl.BlockSpec(s, lambda c: (0,) * len(s))
    wspecs = []
    args = []
    for p in (pt_t, pt_p):
        for k in ('w1', 'b1', 'w2', 'b2', 'w3', 'b3'):
            a = p[k]
            if a.ndim == 1:
                a = a.reshape(1, -1)
            args.append(a)
            wspecs.append(full(a.shape))
    ot, op, d6 = pl.pallas_call(
        _obj_pn_kernel,
        grid=(steps,),
        in_specs=[pl.BlockSpec((_BPC, C, N), lambda c: (c, 0, 0))] + wspecs
                 + [full(cb6.shape)],
        out_specs=[pl.BlockSpec((_BPC, 1, 1024), lambda c: (c, 0, 0)),
                   pl.BlockSpec((_BPC, 1, 1024), lambda c: (c, 0, 0)),
                   pl.BlockSpec((_BPC, 1, 1), lambda c: (c, 0, 0))],
        out_shape=[jax.ShapeDtypeStruct((B, 1, 1024), jnp.float32),
                   jax.ShapeDtypeStruct((B, 1, 1024), jnp.float32),
                   jax.ShapeDtypeStruct((B, 1, 1), jnp.int32)],
    )(obj_pc, *args, cb6)
    return ot.reshape(B, 1024), op.reshape(B, 1024), d6.reshape(B)


_FJ = [0, 1, 2, 3, 4, 5, 5]
_STARTS = [0, 83, 206, 326, 448, 569, 697]
_VALID = [83, 123, 120, 122, 121, 128, 81]


def _hand_pn_kernel(chunk_ref, nat_ref, w1, b1, w2, b2, w3, b3, out_ref):
    j = pl.program_id(0)
    v = jnp.int32(_VALID[-1])
    for jj in range(6):
        v = jnp.where(j == jj, jnp.int32(_VALID[jj]), v)
    nat = nat_ref[...]
    mean = jnp.sum(nat, axis=2) * (1.0 / 778.0)
    B = nat.shape[0]
    x = chunk_ref[0].reshape(B, 128, 3) - mean[:, None, :]
    x = x.reshape(B * 128, 3)
    h = jnp.maximum(_bdot(x, w1[0]) + b1[0], 0.0)
    h = jnp.maximum(_bdot(h, w2[0]) + b2[0], 0.0)
    h = _bdot(h, w3[0]) + b3[0]
    h = h.reshape(B, 128, 1024)
    pid = jax.lax.broadcasted_iota(jnp.int32, (B, 128, 1), 1)
    h = jnp.where(pid < v, h, -1e30)
    pm = jnp.max(h, axis=1)

    @pl.when(j < 6)
    def _():
        out_ref[0] = pm

    @pl.when(j == 6)
    def _():
        out_ref[0] = jnp.maximum(out_ref[0], pm)


def _hand_pointnets(hand_xyz, enc):
    B = hand_xyz.shape[0]
    hp = jnp.transpose(hand_xyz, (0, 2, 1))
    hp = jnp.pad(hp, ((0, 0), (0, 832 - 778), (0, 0)))
    chunks = jnp.stack([hp[:, s:s + 128, :] for s in _STARTS])
    chunks = chunks.reshape(7, B * 128, 3)
    nat = jnp.pad(hand_xyz, ((0, 0), (0, 0), (0, 832 - 778)))

    stk = lambda k: jnp.stack([enc[i][k] for i in range(6)])
    W1, W2, W3 = stk('w1'), stk('w2'), stk('w3')
    B1 = stk('b1')[:, None, :]
    B2 = stk('b2')[:, None, :]
    B3 = stk('b3')[:, None, :]

    wmap = lambda j: (jnp.minimum(j, 5), 0, 0)
    return pl.pallas_call(
        _hand_pn_kernel,
        grid=(7,),
        in_specs=[
            pl.BlockSpec((1, B * 128, 3), lambda j: (j, 0, 0)),
            pl.BlockSpec(nat.shape, lambda j: (0, 0, 0)),
            pl.BlockSpec((1,) + W1.shape[1:], wmap),
            pl.BlockSpec((1,) + B1.shape[1:], wmap),
            pl.BlockSpec((1,) + W2.shape[1:], wmap),
            pl.BlockSpec((1,) + B2.shape[1:], wmap),
            pl.BlockSpec((1,) + W3.shape[1:], wmap),
            pl.BlockSpec((1,) + B3.shape[1:], wmap),
        ],
        out_specs=pl.BlockSpec((1, B, 1024), wmap),
        out_shape=jax.ShapeDtypeStruct((6, B, 1024), jnp.float32),
    )(chunks, nat, W1, B1, W2, B2, W3, B3)


def _emb_vq_kernel(feat_ref, w0, b0, wm, bm, cb_ref, z_ref, d_ref):
    f = feat_ref[0]
    h = jnp.maximum(_bdot(f, w0[0]) + b0[0], 0.0)
    z = _bdot(h, wm[0]) + bm[0]
    cb = cb_ref[0]
    d = (jnp.sum(z * z, axis=1, keepdims=True)
         - 2.0 * _dot_t(z, cb)
         + jnp.sum(cb * cb, axis=1)[None, :])
    B = d.shape[0]
    lane = jax.lax.broadcasted_iota(jnp.int32, (B, 128), 1)
    dmin = jnp.min(d, axis=1, keepdims=True)
    idx = jnp.min(jnp.where(d == dmin, lane, 128), axis=1)
    z_ref[0] = z
    d_ref[0] = (idx + 128 * pl.program_id(0)).reshape(1, B)


def _emb_vq(feat, emb, cbs):
    B = feat.shape[1]
    stk = lambda k: jnp.stack([emb[i][k] for i in range(6)])
    W0, WM = stk('w0'), stk('wm')
    B0 = stk('b0')[:, None, :]
    BM = stk('bm')[:, None, :]
    CB = jnp.stack(cbs)
    bmap = lambda i: (i, 0, 0)
    return pl.pallas_call(
        _emb_vq_kernel,
        grid=(6,),
        in_specs=[
            pl.BlockSpec((1, B, 1024), bmap),
            pl.BlockSpec((1,) + W0.shape[1:], bmap),
            pl.BlockSpec((1,) + B0.shape[1:], bmap),
            pl.BlockSpec((1,) + WM.shape[1:], bmap),
            pl.BlockSpec((1,) + BM.shape[1:], bmap),
            pl.BlockSpec((1,) + CB.shape[1:], bmap),
        ],
        out_specs=[pl.BlockSpec((1, B, 256), bmap),
                   pl.BlockSpec((1, 1, B), bmap)],
        out_shape=[jax.ShapeDtypeStruct((6, B, 256), jnp.float32),
                   jax.ShapeDtypeStruct((6, 1, B), jnp.int32)],
    )(feat, W0, B0, WM, BM, CB)


def _sc_vq(idxFp, cbF):
    mesh = plsc.VectorSubcoreMesh(core_axis_name="c", subcore_axis_name="s")

    @functools.partial(
        pl.kernel, mesh=mesh,
        out_type=jax.ShapeDtypeStruct((256, 256), jnp.float32),
        scratch_types=[pltpu.VMEM((8,), jnp.int32),
                       pltpu.VMEM((8, 256), jnp.float32),
                       pltpu.SemaphoreType.DMA],
    )
    def k(idxF_hbm, cbF_hbm, qFp_hbm, idxv, rows, sem):
        wid = jax.lax.axis_index("s") * 2 + jax.lax.axis_index("c")
        pltpu.sync_copy(idxF_hbm.at[pl.ds(wid * 8, 8)], idxv)
        pltpu.async_copy(cbF_hbm.at[idxv], rows, sem).wait()
        pltpu.sync_copy(rows, qFp_hbm.at[pl.ds(wid * 8, 8)])

    return k(idxFp, cbF)


def _final_kernel(q_ref, z_ref, idx6_ref, cb6_ref, ot_ref, op_ref,
                  d0, db0, d1, db1, d2p, db2p,
                  p0, pb0, p1, pb1, p2p, pb2p, out_ref, loss_ref):
    opos = op_ref[...]
    B = opos.shape[0]
    lane = jax.lax.broadcasted_iota(jnp.int32, (B, 128), 1)
    onehot = (lane == idx6_ref[...]).astype(jnp.float32)
    q6 = _dot(onehot, cb6_ref[...])
    qflat = q_ref[...].reshape(B, 8, 256)[:, :6, :].reshape(6 * B, 256)
    zz = z_ref[...]
    ssq = 0.0
    qs = []
    for i in range(6):
        qi = qflat[32 * i:32 * (i + 1), :]
        qs.append(qi)
        ssq = ssq + jnp.sum((qi - zz[i]) ** 2)
    ssq6 = jnp.sum((q6 - opos) ** 2)
    loss = (1.25 * ssq / (B * 256.0)
            + 3.0 * ssq6 / (B * 1024.0))

    otype = ot_ref[...]
    x1 = _bdot(otype, d0[1536:, :]) + db0[...]
    for i in range(6):
        x1 = x1 + _bdot(qs[i], d0[256 * i:256 * (i + 1), :])
    h = jnp.maximum(x1, 0.0)
    h = jnp.maximum(_bdot(h, d1[...]) + db1[...], 0.0)
    recon = _bdot(h, d2p[...]) + db2p[...]
    y1 = _bdot(q6, p0[:1024, :]) + _bdot(otype, p0[1024:, :]) + pb0[...]
    g = jnp.maximum(y1, 0.0)
    g = jnp.maximum(_bdot(g, p1[...]) + pb1[...], 0.0)
    pos = _bdot(g, p2p[...]) + pb2p[...]
    out_ref[...] = recon + pos
    loss_ref[...] = loss.reshape(1, 1)


def _final(q, z, idx6, cb6, otype, opos, dec, pos_dec):
    B = otype.shape[0]
    d2p = jnp.pad(dec['w2'], ((0, 0), (0, 6)))
    db2p = jnp.pad(dec['b2'], (0, 6)).reshape(1, 61)
    p2p = jnp.pad(pos_dec['w2'], ((0, 0), (55, 0)))
    pb2p = jnp.pad(pos_dec['b2'], (55, 0)).reshape(1, 61)
    args = [q, z, idx6, cb6, otype, opos,
            dec['w0'], dec['b0'].reshape(1, -1),
            dec['w1'], dec['b1'].reshape(1, -1), d2p, db2p,
            pos_dec['w0'], pos_dec['b0'].reshape(1, -1),
            pos_dec['w1'], pos_dec['b1'].reshape(1, -1), p2p, pb2p]
    out, loss = pl.pallas_call(
        _final_kernel,
        in_specs=[pl.BlockSpec(a.shape, functools.partial(lambda n: (0,) * n, a.ndim))
                  for a in args],
        out_specs=[pl.BlockSpec((B, 61), lambda: (0, 0)),
                   pl.BlockSpec((1, 1), lambda: (0, 0))],
        out_shape=[jax.ShapeDtypeStruct((B, 61), jnp.float32),
                   jax.ShapeDtypeStruct((1, 1), jnp.float32)],
    )(*args)
    return out, loss


def kernel(obj_pc, hand_xyz, params):
    feat = _hand_pointnets(hand_xyz, params['hand_enc'])
    z, idxF = _emb_vq(feat, params['emb'], params['cb'])
    cbF = jnp.concatenate(params['cb'], axis=0)
    idxFp = jnp.pad(idxF.reshape(192).reshape(32, 6), ((0, 0), (0, 2)))
    qFp = _sc_vq(idxFp.reshape(256), cbF)
    otype, opos, idx6 = _obj_pointnets(obj_pc, params['obj_type'],
                                       params['obj_pos'], params['cb6'])
    out, loss = _final(qFp, z, idx6.reshape(32, 1), params['cb6'], otype, opos,
                       params['dec'], params['pos_dec'])
    return out, loss[0, 0]


# speedup vs baseline: 1.0514x; 1.0270x over previous
import functools

import jax
import jax.numpy as jnp
from jax.experimental import pallas as pl
from jax.experimental.pallas import tpu as pltpu
from jax.experimental.pallas import tpu_sc as plsc

def _dot(a, b):
    return jnp.dot(a, b, preferred_element_type=jnp.float32)


_bdot = _dot


def _dot_t(a, b):
    return jax.lax.dot_general(a, b, (((1,), (1,)), ((), ())),
                               preferred_element_type=jnp.float32)


_BPC = 2
_NPB = 2048


def _dot0(a, b):
    return jax.lax.dot_general(a, b, (((0,), (0,)), ((), ())),
                               preferred_element_type=jnp.float32)


def _obj_pn_kernel(x_ref, wt1, bt1, wt2, bt2, wt3, bt3,
                   wp1, bp1, wp2, bp2, wp3, bp3, ot_ref, op_ref):
    def chain(xb, w1, b1, w2, b2, w3, b3):
        h = jnp.maximum(_dot0(xb, w1[...]) + b1[...], 0.0)
        h = jnp.maximum(_bdot(h, w2[...]) + b2[...], 0.0)
        h = _bdot(h, w3[...]) + b3[...]
        return jnp.max(h, axis=0, keepdims=True)

    pts, pps = [], []
    for i in range(_BPC):
        xb = x_ref[i]
        pts.append(chain(xb, wt1, bt1, wt2, bt2, wt3, bt3))
        pps.append(chain(xb, wp1, bp1, wp2, bp2, wp3, bp3))
    ot_ref[...] = jnp.stack(pts)
    op_ref[...] = jnp.stack(pps)


def _obj_pointnets(obj_pc, pt_t, pt_p):
    B, C, N = obj_pc.shape
    steps = B // _BPC
    full = lambda s: pl.BlockSpec(s, lambda c: (0,) * len(s))
    wspecs = []
    args = []
    for p in (pt_t, pt_p):
        for k in ('w1', 'b1', 'w2', 'b2', 'w3', 'b3'):
            a = p[k]
            if a.ndim == 1:
                a = a.reshape(1, -1)
            args.append(a)
            wspecs.append(full(a.shape))
    ot, op = pl.pallas_call(
        _obj_pn_kernel,
        grid=(steps,),
        in_specs=[pl.BlockSpec((_BPC, C, N), lambda c: (c, 0, 0))] + wspecs,
        out_specs=[pl.BlockSpec((_BPC, 1, 1024), lambda c: (c, 0, 0)),
                   pl.BlockSpec((_BPC, 1, 1024), lambda c: (c, 0, 0))],
        out_shape=[jax.ShapeDtypeStruct((B, 1, 1024), jnp.float32),
                   jax.ShapeDtypeStruct((B, 1, 1024), jnp.float32)],
    )(obj_pc, *args)
    return ot.reshape(B, 1024), op.reshape(B, 1024)


_FJ = [0, 1, 2, 3, 4, 5, 5]
_STARTS = [0, 83, 206, 326, 448, 569, 697]
_VALID = [83, 123, 120, 122, 121, 128, 81]


def _hand_pn_kernel(chunk_ref, nat_ref, w1, b1, w2, b2, w3, b3, out_ref):
    j = pl.program_id(0)
    v = jnp.int32(_VALID[-1])
    for jj in range(6):
        v = jnp.where(j == jj, jnp.int32(_VALID[jj]), v)
    nat = nat_ref[...]
    mean = jnp.sum(nat, axis=2) * (1.0 / 778.0)
    B = nat.shape[0]
    x = chunk_ref[0].reshape(B, 128, 3) - mean[:, None, :]
    x = x.reshape(B * 128, 3)
    h = jnp.maximum(_bdot(x, w1[0]) + b1[0], 0.0)
    h = jnp.maximum(_bdot(h, w2[0]) + b2[0], 0.0)
    h = _bdot(h, w3[0]) + b3[0]
    h = h.reshape(B, 128, 1024)
    pid = jax.lax.broadcasted_iota(jnp.int32, (B, 128, 1), 1)
    h = jnp.where(pid < v, h, -1e30)
    pm = jnp.max(h, axis=1)

    @pl.when(j < 6)
    def _():
        out_ref[0] = pm

    @pl.when(j == 6)
    def _():
        out_ref[0] = jnp.maximum(out_ref[0], pm)


def _hand_pointnets(hand_xyz, enc):
    B = hand_xyz.shape[0]
    hp = jnp.transpose(hand_xyz, (0, 2, 1))
    hp = jnp.pad(hp, ((0, 0), (0, 832 - 778), (0, 0)))
    chunks = jnp.stack([hp[:, s:s + 128, :] for s in _STARTS])
    chunks = chunks.reshape(7, B * 128, 3)
    nat = jnp.pad(hand_xyz, ((0, 0), (0, 0), (0, 832 - 778)))

    stk = lambda k: jnp.stack([enc[i][k] for i in range(6)])
    W1, W2, W3 = stk('w1'), stk('w2'), stk('w3')
    B1 = stk('b1')[:, None, :]
    B2 = stk('b2')[:, None, :]
    B3 = stk('b3')[:, None, :]

    wmap = lambda j: (jnp.minimum(j, 5), 0, 0)
    return pl.pallas_call(
        _hand_pn_kernel,
        grid=(7,),
        in_specs=[
            pl.BlockSpec((1, B * 128, 3), lambda j: (j, 0, 0)),
            pl.BlockSpec(nat.shape, lambda j: (0, 0, 0)),
            pl.BlockSpec((1,) + W1.shape[1:], wmap),
            pl.BlockSpec((1,) + B1.shape[1:], wmap),
            pl.BlockSpec((1,) + W2.shape[1:], wmap),
            pl.BlockSpec((1,) + B2.shape[1:], wmap),
            pl.BlockSpec((1,) + W3.shape[1:], wmap),
            pl.BlockSpec((1,) + B3.shape[1:], wmap),
        ],
        out_specs=pl.BlockSpec((1, B, 1024), wmap),
        out_shape=jax.ShapeDtypeStruct((6, B, 1024), jnp.float32),
    )(chunks, nat, W1, B1, W2, B2, W3, B3)


def _emb_vq_kernel(feat_ref, w0, b0, wm, bm, cb_ref, z_ref, d_ref):
    f = feat_ref[0]
    h = jnp.maximum(_bdot(f, w0[0]) + b0[0], 0.0)
    z = _bdot(h, wm[0]) + bm[0]
    cb = cb_ref[0]
    d = (jnp.sum(z * z, axis=1, keepdims=True)
         - 2.0 * _dot_t(z, cb)
         + jnp.sum(cb * cb, axis=1)[None, :])
    B = d.shape[0]
    lane = jax.lax.broadcasted_iota(jnp.int32, (B, 128), 1)
    dmin = jnp.min(d, axis=1, keepdims=True)
    idx = jnp.min(jnp.where(d == dmin, lane, 128), axis=1)
    z_ref[0] = z
    d_ref[0] = (idx + 128 * pl.program_id(0)).reshape(1, B)


def _emb_vq(feat, emb, cbs):
    B = feat.shape[1]
    stk = lambda k: jnp.stack([emb[i][k] for i in range(6)])
    W0, WM = stk('w0'), stk('wm')
    B0 = stk('b0')[:, None, :]
    BM = stk('bm')[:, None, :]
    CB = jnp.stack(cbs)
    bmap = lambda i: (i, 0, 0)
    return pl.pallas_call(
        _emb_vq_kernel,
        grid=(6,),
        in_specs=[
            pl.BlockSpec((1, B, 1024), bmap),
            pl.BlockSpec((1,) + W0.shape[1:], bmap),
            pl.BlockSpec((1,) + B0.shape[1:], bmap),
            pl.BlockSpec((1,) + WM.shape[1:], bmap),
            pl.BlockSpec((1,) + BM.shape[1:], bmap),
            pl.BlockSpec((1,) + CB.shape[1:], bmap),
        ],
        out_specs=[pl.BlockSpec((1, B, 256), bmap),
                   pl.BlockSpec((1, 1, B), bmap)],
        out_shape=[jax.ShapeDtypeStruct((6, B, 256), jnp.float32),
                   jax.ShapeDtypeStruct((6, 1, B), jnp.int32)],
    )(feat, W0, B0, WM, BM, CB)


def _sc_vq(idxFp, cbF):
    mesh = plsc.VectorSubcoreMesh(core_axis_name="c", subcore_axis_name="s")

    @functools.partial(
        pl.kernel, mesh=mesh,
        out_type=jax.ShapeDtypeStruct((256, 256), jnp.float32),
        scratch_types=[pltpu.VMEM((8,), jnp.int32),
                       pltpu.VMEM((8, 256), jnp.float32),
                       pltpu.SemaphoreType.DMA],
    )
    def k(idxF_hbm, cbF_hbm, qFp_hbm, idxv, rows, sem):
        wid = jax.lax.axis_index("s") * 2 + jax.lax.axis_index("c")
        pltpu.sync_copy(idxF_hbm.at[pl.ds(wid * 8, 8)], idxv)
        pltpu.async_copy(cbF_hbm.at[idxv], rows, sem).wait()
        pltpu.sync_copy(rows, qFp_hbm.at[pl.ds(wid * 8, 8)])

    return k(idxFp, cbF)


def _final_kernel(q_ref, z_ref, cb6_ref, ot_ref, op_ref,
                  d0, db0, d1, db1, d2p, db2p,
                  p0, pb0, p1, pb1, p2p, pb2p, out_ref, loss_ref):
    opos = op_ref[...]
    B = opos.shape[0]
    cb6 = cb6_ref[...]
    d6 = (jnp.sum(opos * opos, axis=1, keepdims=True)
          - 2.0 * _dot_t(opos, cb6)
          + jnp.sum(cb6 * cb6, axis=1)[None, :])
    lane = jax.lax.broadcasted_iota(jnp.int32, (B, 128), 1)
    dmin = jnp.min(d6, axis=1, keepdims=True)
    idx6 = jnp.min(jnp.where(d6 == dmin, lane, 128), axis=1, keepdims=True)
    onehot = (lane == idx6).astype(jnp.float32)
    q6 = _dot(onehot, cb6)
    qflat = q_ref[...].reshape(B, 8, 256)[:, :6, :].reshape(6 * B, 256)
    zz = z_ref[...]
    ssq = 0.0
    qs = []
    for i in range(6):
        qi = qflat[32 * i:32 * (i + 1), :]
        qs.append(qi)
        ssq = ssq + jnp.sum((qi - zz[i]) ** 2)
    ssq6 = jnp.sum((q6 - opos) ** 2)
    loss = (1.25 * ssq / (B * 256.0)
            + 3.0 * ssq6 / (B * 1024.0))

    otype = ot_ref[...]
    x1 = _bdot(otype, d0[1536:, :]) + db0[...]
    for i in range(6):
        x1 = x1 + _bdot(qs[i], d0[256 * i:256 * (i + 1), :])
    h = jnp.maximum(x1, 0.0)
    h = jnp.maximum(_bdot(h, d1[...]) + db1[...], 0.0)
    recon = _bdot(h, d2p[...]) + db2p[...]
    y1 = _bdot(q6, p0[:1024, :]) + _bdot(otype, p0[1024:, :]) + pb0[...]
    g = jnp.maximum(y1, 0.0)
    g = jnp.maximum(_bdot(g, p1[...]) + pb1[...], 0.0)
    pos = _bdot(g, p2p[...]) + pb2p[...]
    out_ref[...] = recon + pos
    loss_ref[...] = loss.reshape(1, 1)


def _final(q, z, cb6, otype, opos, dec, pos_dec):
    B = otype.shape[0]
    d2p = jnp.pad(dec['w2'], ((0, 0), (0, 6)))
    db2p = jnp.pad(dec['b2'], (0, 6)).reshape(1, 61)
    p2p = jnp.pad(pos_dec['w2'], ((0, 0), (55, 0)))
    pb2p = jnp.pad(pos_dec['b2'], (55, 0)).reshape(1, 61)
    args = [q, z, cb6, otype, opos,
            dec['w0'], dec['b0'].reshape(1, -1),
            dec['w1'], dec['b1'].reshape(1, -1), d2p, db2p,
            pos_dec['w0'], pos_dec['b0'].reshape(1, -1),
            pos_dec['w1'], pos_dec['b1'].reshape(1, -1), p2p, pb2p]
    out, loss = pl.pallas_call(
        _final_kernel,
        in_specs=[pl.BlockSpec(a.shape, functools.partial(lambda n: (0,) * n, a.ndim))
                  for a in args],
        out_specs=[pl.BlockSpec((B, 61), lambda: (0, 0)),
                   pl.BlockSpec((1, 1), lambda: (0, 0))],
        out_shape=[jax.ShapeDtypeStruct((B, 61), jnp.float32),
                   jax.ShapeDtypeStruct((1, 1), jnp.float32)],
    )(*args)
    return out, loss


def kernel(obj_pc, hand_xyz, params):
    feat = _hand_pointnets(hand_xyz, params['hand_enc'])
    z, idxF = _emb_vq(feat, params['emb'], params['cb'])
    cbF = jnp.concatenate(params['cb'], axis=0)
    idxFp = jnp.pad(idxF.reshape(192).reshape(32, 6), ((0, 0), (0, 2)))
    qFp = _sc_vq(idxFp.reshape(256), cbF)
    otype, opos = _obj_pointnets(obj_pc, params['obj_type'],
                                 params['obj_pos'])
    out, loss = _final(qFp, z, params['cb6'], otype, opos,
                       params['dec'], params['pos_dec'])
    return out, loss[0, 0]


# speedup vs baseline: 1.0675x; 1.0153x over previous
import functools

import jax
import jax.numpy as jnp
from jax.experimental import pallas as pl
from jax.experimental.pallas import tpu as pltpu
from jax.experimental.pallas import tpu_sc as plsc

def _dot(a, b):
    return jnp.dot(a, b, preferred_element_type=jnp.float32)


_bdot = _dot


def _dot_t(a, b):
    return jax.lax.dot_general(a, b, (((1,), (1,)), ((), ())),
                               preferred_element_type=jnp.float32)


_BPC = 4
_NPB = 2048


def _dot0(a, b):
    return jax.lax.dot_general(a, b, (((0,), (0,)), ((), ())),
                               preferred_element_type=jnp.float32)


def _obj_pn_kernel(x_ref, wt1, bt1, wt2, bt2, wt3, bt3,
                   wp1, bp1, wp2, bp2, wp3, bp3, ot_ref, op_ref):
    def chain(xb, w1, b1, w2, b2, w3, b3):
        h = jnp.maximum(_dot0(xb, w1[...]) + b1[...], 0.0)
        h = jnp.maximum(_bdot(h, w2[...]) + b2[...], 0.0)
        h = _bdot(h, w3[...]) + b3[...]
        return jnp.max(h, axis=0, keepdims=True)

    pts, pps = [], []
    for i in range(_BPC):
        xb = x_ref[i]
        pts.append(chain(xb, wt1, bt1, wt2, bt2, wt3, bt3))
        pps.append(chain(xb, wp1, bp1, wp2, bp2, wp3, bp3))
    ot_ref[...] = jnp.stack(pts)
    op_ref[...] = jnp.stack(pps)


def _obj_pointnets(obj_pc, pt_t, pt_p):
    B, C, N = obj_pc.shape
    steps = B // _BPC
    full = lambda s: pl.BlockSpec(s, lambda c: (0,) * len(s))
    wspecs = []
    args = []
    for p in (pt_t, pt_p):
        for k in ('w1', 'b1', 'w2', 'b2', 'w3', 'b3'):
            a = p[k]
            if a.ndim == 1:
                a = a.reshape(1, -1)
            args.append(a)
            wspecs.append(full(a.shape))
    ot, op = pl.pallas_call(
        _obj_pn_kernel,
        grid=(steps,),
        in_specs=[pl.BlockSpec((_BPC, C, N), lambda c: (c, 0, 0))] + wspecs,
        out_specs=[pl.BlockSpec((_BPC, 1, 1024), lambda c: (c, 0, 0)),
                   pl.BlockSpec((_BPC, 1, 1024), lambda c: (c, 0, 0))],
        out_shape=[jax.ShapeDtypeStruct((B, 1, 1024), jnp.float32),
                   jax.ShapeDtypeStruct((B, 1, 1024), jnp.float32)],
    )(obj_pc, *args)
    return ot.reshape(B, 1024), op.reshape(B, 1024)


_FJ = [0, 1, 2, 3, 4, 5, 5]
_STARTS = [0, 83, 206, 326, 448, 569, 697]
_VALID = [83, 123, 120, 122, 121, 128, 81]


def _hand_pn_kernel(chunk_ref, nat_ref, w1, b1, w2, b2, w3, b3, out_ref):
    j = pl.program_id(0)
    v = jnp.int32(_VALID[-1])
    for jj in range(6):
        v = jnp.where(j == jj, jnp.int32(_VALID[jj]), v)
    nat = nat_ref[...]
    mean = jnp.sum(nat, axis=2) * (1.0 / 778.0)
    B = nat.shape[0]
    x = chunk_ref[0].reshape(B, 128, 3) - mean[:, None, :]
    x = x.reshape(B * 128, 3)
    h = jnp.maximum(_bdot(x, w1[0]) + b1[0], 0.0)
    h = jnp.maximum(_bdot(h, w2[0]) + b2[0], 0.0)
    h = _bdot(h, w3[0]) + b3[0]
    h = h.reshape(B, 128, 1024)
    pid = jax.lax.broadcasted_iota(jnp.int32, (B, 128, 1), 1)
    h = jnp.where(pid < v, h, -1e30)
    pm = jnp.max(h, axis=1)

    @pl.when(j < 6)
    def _():
        out_ref[0] = pm

    @pl.when(j == 6)
    def _():
        out_ref[0] = jnp.maximum(out_ref[0], pm)


def _hand_pointnets(hand_xyz, enc):
    B = hand_xyz.shape[0]
    hp = jnp.transpose(hand_xyz, (0, 2, 1))
    hp = jnp.pad(hp, ((0, 0), (0, 832 - 778), (0, 0)))
    chunks = jnp.stack([hp[:, s:s + 128, :] for s in _STARTS])
    chunks = chunks.reshape(7, B * 128, 3)
    nat = jnp.pad(hand_xyz, ((0, 0), (0, 0), (0, 832 - 778)))

    stk = lambda k: jnp.stack([enc[i][k] for i in range(6)])
    W1, W2, W3 = stk('w1'), stk('w2'), stk('w3')
    B1 = stk('b1')[:, None, :]
    B2 = stk('b2')[:, None, :]
    B3 = stk('b3')[:, None, :]

    wmap = lambda j: (jnp.minimum(j, 5), 0, 0)
    return pl.pallas_call(
        _hand_pn_kernel,
        grid=(7,),
        in_specs=[
            pl.BlockSpec((1, B * 128, 3), lambda j: (j, 0, 0)),
            pl.BlockSpec(nat.shape, lambda j: (0, 0, 0)),
            pl.BlockSpec((1,) + W1.shape[1:], wmap),
            pl.BlockSpec((1,) + B1.shape[1:], wmap),
            pl.BlockSpec((1,) + W2.shape[1:], wmap),
            pl.BlockSpec((1,) + B2.shape[1:], wmap),
            pl.BlockSpec((1,) + W3.shape[1:], wmap),
            pl.BlockSpec((1,) + B3.shape[1:], wmap),
        ],
        out_specs=pl.BlockSpec((1, B, 1024), wmap),
        out_shape=jax.ShapeDtypeStruct((6, B, 1024), jnp.float32),
    )(chunks, nat, W1, B1, W2, B2, W3, B3)


def _emb_vq_kernel(feat_ref, w0, b0, wm, bm, cb_ref, z_ref, d_ref):
    f = feat_ref[0]
    h = jnp.maximum(_bdot(f, w0[0]) + b0[0], 0.0)
    z = _bdot(h, wm[0]) + bm[0]
    cb = cb_ref[0]
    d = (jnp.sum(z * z, axis=1, keepdims=True)
         - 2.0 * _dot_t(z, cb)
         + jnp.sum(cb * cb, axis=1)[None, :])
    B = d.shape[0]
    lane = jax.lax.broadcasted_iota(jnp.int32, (B, 128), 1)
    dmin = jnp.min(d, axis=1, keepdims=True)
    idx = jnp.min(jnp.where(d == dmin, lane, 128), axis=1)
    z_ref[0] = z
    d_ref[0] = (idx + 128 * pl.program_id(0)).reshape(1, B)


def _emb_vq(feat, emb, cbs):
    B = feat.shape[1]
    stk = lambda k: jnp.stack([emb[i][k] for i in range(6)])
    W0, WM = stk('w0'), stk('wm')
    B0 = stk('b0')[:, None, :]
    BM = stk('bm')[:, None, :]
    CB = jnp.stack(cbs)
    bmap = lambda i: (i, 0, 0)
    return pl.pallas_call(
        _emb_vq_kernel,
        grid=(6,),
        in_specs=[
            pl.BlockSpec((1, B, 1024), bmap),
            pl.BlockSpec((1,) + W0.shape[1:], bmap),
            pl.BlockSpec((1,) + B0.shape[1:], bmap),
            pl.BlockSpec((1,) + WM.shape[1:], bmap),
            pl.BlockSpec((1,) + BM.shape[1:], bmap),
            pl.BlockSpec((1,) + CB.shape[1:], bmap),
        ],
        out_specs=[pl.BlockSpec((1, B, 256), bmap),
                   pl.BlockSpec((1, 1, B), bmap)],
        out_shape=[jax.ShapeDtypeStruct((6, B, 256), jnp.float32),
                   jax.ShapeDtypeStruct((6, 1, B), jnp.int32)],
    )(feat, W0, B0, WM, BM, CB)


def _sc_vq(idxFp, cbF):
    mesh = plsc.VectorSubcoreMesh(core_axis_name="c", subcore_axis_name="s")

    @functools.partial(
        pl.kernel, mesh=mesh,
        out_type=jax.ShapeDtypeStruct((256, 256), jnp.float32),
        scratch_types=[pltpu.VMEM((8,), jnp.int32),
                       pltpu.VMEM((8, 256), jnp.float32),
                       pltpu.SemaphoreType.DMA],
    )
    def k(idxF_hbm, cbF_hbm, qFp_hbm, idxv, rows, sem):
        wid = jax.lax.axis_index("s") * 2 + jax.lax.axis_index("c")
        pltpu.sync_copy(idxF_hbm.at[pl.ds(wid * 8, 8)], idxv)
        pltpu.async_copy(cbF_hbm.at[idxv], rows, sem).wait()
        pltpu.sync_copy(rows, qFp_hbm.at[pl.ds(wid * 8, 8)])

    return k(idxFp, cbF)


def _final_kernel(q_ref, z_ref, cb6_ref, ot_ref, op_ref,
                  d0, db0, d1, db1, d2p, db2p,
                  p0, pb0, p1, pb1, p2p, pb2p, out_ref, loss_ref):
    opos = op_ref[...]
    B = opos.shape[0]
    cb6 = cb6_ref[...]
    d6 = (jnp.sum(opos * opos, axis=1, keepdims=True)
          - 2.0 * _dot_t(opos, cb6)
          + jnp.sum(cb6 * cb6, axis=1)[None, :])
    lane = jax.lax.broadcasted_iota(jnp.int32, (B, 128), 1)
    dmin = jnp.min(d6, axis=1, keepdims=True)
    idx6 = jnp.min(jnp.where(d6 == dmin, lane, 128), axis=1, keepdims=True)
    onehot = (lane == idx6).astype(jnp.float32)
    q6 = _dot(onehot, cb6)
    qflat = q_ref[...].reshape(B, 8, 256)[:, :6, :].reshape(6 * B, 256)
    zz = z_ref[...]
    ssq = 0.0
    qs = []
    for i in range(6):
        qi = qflat[32 * i:32 * (i + 1), :]
        qs.append(qi)
        ssq = ssq + jnp.sum((qi - zz[i]) ** 2)
    ssq6 = jnp.sum((q6 - opos) ** 2)
    loss = (1.25 * ssq / (B * 256.0)
            + 3.0 * ssq6 / (B * 1024.0))

    otype = ot_ref[...]
    x1 = _bdot(otype, d0[1536:, :]) + db0[...]
    for i in range(6):
        x1 = x1 + _bdot(qs[i], d0[256 * i:256 * (i + 1), :])
    h = jnp.maximum(x1, 0.0)
    h = jnp.maximum(_bdot(h, d1[...]) + db1[...], 0.0)
    recon = _bdot(h, d2p[...]) + db2p[...]
    y1 = _bdot(q6, p0[:1024, :]) + _bdot(otype, p0[1024:, :]) + pb0[...]
    g = jnp.maximum(y1, 0.0)
    g = jnp.maximum(_bdot(g, p1[...]) + pb1[...], 0.0)
    pos = _bdot(g, p2p[...]) + pb2p[...]
    out_ref[...] = recon + pos
    loss_ref[...] = loss.reshape(1, 1)


def _final(q, z, cb6, otype, opos, dec, pos_dec):
    B = otype.shape[0]
    d2p = jnp.pad(dec['w2'], ((0, 0), (0, 6)))
    db2p = jnp.pad(dec['b2'], (0, 6)).reshape(1, 61)
    p2p = jnp.pad(pos_dec['w2'], ((0, 0), (55, 0)))
    pb2p = jnp.pad(pos_dec['b2'], (55, 0)).reshape(1, 61)
    args = [q, z, cb6, otype, opos,
            dec['w0'], dec['b0'].reshape(1, -1),
            dec['w1'], dec['b1'].reshape(1, -1), d2p, db2p,
            pos_dec['w0'], pos_dec['b0'].reshape(1, -1),
            pos_dec['w1'], pos_dec['b1'].reshape(1, -1), p2p, pb2p]
    out, loss = pl.pallas_call(
        _final_kernel,
        in_specs=[pl.BlockSpec(a.shape, functools.partial(lambda n: (0,) * n, a.ndim))
                  for a in args],
        out_specs=[pl.BlockSpec((B, 61), lambda: (0, 0)),
                   pl.BlockSpec((1, 1), lambda: (0, 0))],
        out_shape=[jax.ShapeDtypeStruct((B, 61), jnp.float32),
                   jax.ShapeDtypeStruct((1, 1), jnp.float32)],
    )(*args)
    return out, loss


def kernel(obj_pc, hand_xyz, params):
    feat = _hand_pointnets(hand_xyz, params['hand_enc'])
    z, idxF = _emb_vq(feat, params['emb'], params['cb'])
    cbF = jnp.concatenate(params['cb'], axis=0)
    idxFp = jnp.pad(idxF.reshape(192).reshape(32, 6), ((0, 0), (0, 2)))
    qFp = _sc_vq(idxFp.reshape(256), cbF)
    otype, opos = _obj_pointnets(obj_pc, params['obj_type'],
                                 params['obj_pos'])
    out, loss = _final(qFp, z, params['cb6'], otype, opos,
                       params['dec'], params['pos_dec'])
    return out, loss[0, 0]


# speedup vs baseline: 1.0744x; 1.0064x over previous
import functools

import jax
import jax.numpy as jnp
from jax.experimental import pallas as pl
from jax.experimental.pallas import tpu as pltpu
from jax.experimental.pallas import tpu_sc as plsc

def _dot(a, b):
    return jnp.dot(a, b, preferred_element_type=jnp.float32)


_bdot = _dot


def _dot_t(a, b):
    return jax.lax.dot_general(a, b, (((1,), (1,)), ((), ())),
                               preferred_element_type=jnp.float32)


_BPC = 8
_NPB = 2048


def _dot0(a, b):
    return jax.lax.dot_general(a, b, (((0,), (0,)), ((), ())),
                               preferred_element_type=jnp.float32)


def _obj_pn_kernel(x_ref, wt1, bt1, wt2, bt2, wt3, bt3,
                   wp1, bp1, wp2, bp2, wp3, bp3, ot_ref, op_ref):
    def chain(xb, w1, b1, w2, b2, w3, b3):
        h = jnp.maximum(_dot0(xb, w1[...]) + b1[...], 0.0)
        h = jnp.maximum(_bdot(h, w2[...]) + b2[...], 0.0)
        h = _bdot(h, w3[...]) + b3[...]
        return jnp.max(h, axis=0, keepdims=True)

    pts, pps = [], []
    for i in range(_BPC):
        xb = x_ref[i]
        pts.append(chain(xb, wt1, bt1, wt2, bt2, wt3, bt3))
        pps.append(chain(xb, wp1, bp1, wp2, bp2, wp3, bp3))
    ot_ref[...] = jnp.stack(pts)
    op_ref[...] = jnp.stack(pps)


def _obj_pointnets(obj_pc, pt_t, pt_p):
    B, C, N = obj_pc.shape
    steps = B // _BPC
    full = lambda s: pl.BlockSpec(s, lambda c: (0,) * len(s))
    wspecs = []
    args = []
    for p in (pt_t, pt_p):
        for k in ('w1', 'b1', 'w2', 'b2', 'w3', 'b3'):
            a = p[k]
            if a.ndim == 1:
                a = a.reshape(1, -1)
            args.append(a)
            wspecs.append(full(a.shape))
    ot, op = pl.pallas_call(
        _obj_pn_kernel,
        grid=(steps,),
        in_specs=[pl.BlockSpec((_BPC, C, N), lambda c: (c, 0, 0))] + wspecs,
        out_specs=[pl.BlockSpec((_BPC, 1, 1024), lambda c: (c, 0, 0)),
                   pl.BlockSpec((_BPC, 1, 1024), lambda c: (c, 0, 0))],
        out_shape=[jax.ShapeDtypeStruct((B, 1, 1024), jnp.float32),
                   jax.ShapeDtypeStruct((B, 1, 1024), jnp.float32)],
    )(obj_pc, *args)
    return ot.reshape(B, 1024), op.reshape(B, 1024)


_FJ = [0, 1, 2, 3, 4, 5, 5]
_STARTS = [0, 83, 206, 326, 448, 569, 697]
_VALID = [83, 123, 120, 122, 121, 128, 81]


def _hand_pn_kernel(chunk_ref, nat_ref, w1, b1, w2, b2, w3, b3, out_ref):
    j = pl.program_id(0)
    v = jnp.int32(_VALID[-1])
    for jj in range(6):
        v = jnp.where(j == jj, jnp.int32(_VALID[jj]), v)
    nat = nat_ref[...]
    mean = jnp.sum(nat, axis=2) * (1.0 / 778.0)
    B = nat.shape[0]
    x = chunk_ref[0].reshape(B, 128, 3) - mean[:, None, :]
    x = x.reshape(B * 128, 3)
    h = jnp.maximum(_bdot(x, w1[0]) + b1[0], 0.0)
    h = jnp.maximum(_bdot(h, w2[0]) + b2[0], 0.0)
    h = _bdot(h, w3[0]) + b3[0]
    h = h.reshape(B, 128, 1024)
    pid = jax.lax.broadcasted_iota(jnp.int32, (B, 128, 1), 1)
    h = jnp.where(pid < v, h, -1e30)
    pm = jnp.max(h, axis=1)

    @pl.when(j < 6)
    def _():
        out_ref[0] = pm

    @pl.when(j == 6)
    def _():
        out_ref[0] = jnp.maximum(out_ref[0], pm)


def _hand_pointnets(hand_xyz, enc):
    B = hand_xyz.shape[0]
    hp = jnp.transpose(hand_xyz, (0, 2, 1))
    hp = jnp.pad(hp, ((0, 0), (0, 832 - 778), (0, 0)))
    chunks = jnp.stack([hp[:, s:s + 128, :] for s in _STARTS])
    chunks = chunks.reshape(7, B * 128, 3)
    nat = jnp.pad(hand_xyz, ((0, 0), (0, 0), (0, 832 - 778)))

    stk = lambda k: jnp.stack([enc[i][k] for i in range(6)])
    W1, W2, W3 = stk('w1'), stk('w2'), stk('w3')
    B1 = stk('b1')[:, None, :]
    B2 = stk('b2')[:, None, :]
    B3 = stk('b3')[:, None, :]

    wmap = lambda j: (jnp.minimum(j, 5), 0, 0)
    return pl.pallas_call(
        _hand_pn_kernel,
        grid=(7,),
        in_specs=[
            pl.BlockSpec((1, B * 128, 3), lambda j: (j, 0, 0)),
            pl.BlockSpec(nat.shape, lambda j: (0, 0, 0)),
            pl.BlockSpec((1,) + W1.shape[1:], wmap),
            pl.BlockSpec((1,) + B1.shape[1:], wmap),
            pl.BlockSpec((1,) + W2.shape[1:], wmap),
            pl.BlockSpec((1,) + B2.shape[1:], wmap),
            pl.BlockSpec((1,) + W3.shape[1:], wmap),
            pl.BlockSpec((1,) + B3.shape[1:], wmap),
        ],
        out_specs=pl.BlockSpec((1, B, 1024), wmap),
        out_shape=jax.ShapeDtypeStruct((6, B, 1024), jnp.float32),
    )(chunks, nat, W1, B1, W2, B2, W3, B3)


def _emb_vq_kernel(feat_ref, w0, b0, wm, bm, cb_ref, z_ref, d_ref):
    f = feat_ref[0]
    h = jnp.maximum(_bdot(f, w0[0]) + b0[0], 0.0)
    z = _bdot(h, wm[0]) + bm[0]
    cb = cb_ref[0]
    d = (jnp.sum(z * z, axis=1, keepdims=True)
         - 2.0 * _dot_t(z, cb)
         + jnp.sum(cb * cb, axis=1)[None, :])
    B = d.shape[0]
    lane = jax.lax.broadcasted_iota(jnp.int32, (B, 128), 1)
    dmin = jnp.min(d, axis=1, keepdims=True)
    idx = jnp.min(jnp.where(d == dmin, lane, 128), axis=1)
    z_ref[0] = z
    d_ref[0] = (idx + 128 * pl.program_id(0)).reshape(1, B)


def _emb_vq(feat, emb, cbs):
    B = feat.shape[1]
    stk = lambda k: jnp.stack([emb[i][k] for i in range(6)])
    W0, WM = stk('w0'), stk('wm')
    B0 = stk('b0')[:, None, :]
    BM = stk('bm')[:, None, :]
    CB = jnp.stack(cbs)
    bmap = lambda i: (i, 0, 0)
    return pl.pallas_call(
        _emb_vq_kernel,
        grid=(6,),
        in_specs=[
            pl.BlockSpec((1, B, 1024), bmap),
            pl.BlockSpec((1,) + W0.shape[1:], bmap),
            pl.BlockSpec((1,) + B0.shape[1:], bmap),
            pl.BlockSpec((1,) + WM.shape[1:], bmap),
            pl.BlockSpec((1,) + BM.shape[1:], bmap),
            pl.BlockSpec((1,) + CB.shape[1:], bmap),
        ],
        out_specs=[pl.BlockSpec((1, B, 256), bmap),
                   pl.BlockSpec((1, 1, B), bmap)],
        out_shape=[jax.ShapeDtypeStruct((6, B, 256), jnp.float32),
                   jax.ShapeDtypeStruct((6, 1, B), jnp.int32)],
    )(feat, W0, B0, WM, BM, CB)


def _sc_vq(idxFp, cbF):
    mesh = plsc.VectorSubcoreMesh(core_axis_name="c", subcore_axis_name="s")

    @functools.partial(
        pl.kernel, mesh=mesh,
        out_type=jax.ShapeDtypeStruct((256, 256), jnp.float32),
        scratch_types=[pltpu.VMEM((8,), jnp.int32),
                       pltpu.VMEM((8, 256), jnp.float32),
                       pltpu.SemaphoreType.DMA],
    )
    def k(idxF_hbm, cbF_hbm, qFp_hbm, idxv, rows, sem):
        wid = jax.lax.axis_index("s") * 2 + jax.lax.axis_index("c")
        pltpu.sync_copy(idxF_hbm.at[pl.ds(wid * 8, 8)], idxv)
        pltpu.async_copy(cbF_hbm.at[idxv], rows, sem).wait()
        pltpu.sync_copy(rows, qFp_hbm.at[pl.ds(wid * 8, 8)])

    return k(idxFp, cbF)


def _final_kernel(q_ref, z_ref, cb6_ref, ot_ref, op_ref,
                  d0, db0, d1, db1, d2p, db2p,
                  p0, pb0, p1, pb1, p2p, pb2p, out_ref, loss_ref):
    opos = op_ref[...]
    B = opos.shape[0]
    cb6 = cb6_ref[...]
    d6 = (jnp.sum(opos * opos, axis=1, keepdims=True)
          - 2.0 * _dot_t(opos, cb6)
          + jnp.sum(cb6 * cb6, axis=1)[None, :])
    lane = jax.lax.broadcasted_iota(jnp.int32, (B, 128), 1)
    dmin = jnp.min(d6, axis=1, keepdims=True)
    idx6 = jnp.min(jnp.where(d6 == dmin, lane, 128), axis=1, keepdims=True)
    onehot = (lane == idx6).astype(jnp.float32)
    q6 = _dot(onehot, cb6)
    qflat = q_ref[...].reshape(B, 8, 256)[:, :6, :].reshape(6 * B, 256)
    zz = z_ref[...]
    ssq = 0.0
    qs = []
    for i in range(6):
        qi = qflat[32 * i:32 * (i + 1), :]
        qs.append(qi)
        ssq = ssq + jnp.sum((qi - zz[i]) ** 2)
    ssq6 = jnp.sum((q6 - opos) ** 2)
    loss = (1.25 * ssq / (B * 256.0)
            + 3.0 * ssq6 / (B * 1024.0))

    otype = ot_ref[...]
    x1 = _bdot(otype, d0[1536:, :]) + db0[...]
    for i in range(6):
        x1 = x1 + _bdot(qs[i], d0[256 * i:256 * (i + 1), :])
    h = jnp.maximum(x1, 0.0)
    h = jnp.maximum(_bdot(h, d1[...]) + db1[...], 0.0)
    recon = _bdot(h, d2p[...]) + db2p[...]
    y1 = _bdot(q6, p0[:1024, :]) + _bdot(otype, p0[1024:, :]) + pb0[...]
    g = jnp.maximum(y1, 0.0)
    g = jnp.maximum(_bdot(g, p1[...]) + pb1[...], 0.0)
    pos = _bdot(g, p2p[...]) + pb2p[...]
    out_ref[...] = recon + pos
    loss_ref[...] = loss.reshape(1, 1)


def _final(q, z, cb6, otype, opos, dec, pos_dec):
    B = otype.shape[0]
    d2p = jnp.pad(dec['w2'], ((0, 0), (0, 6)))
    db2p = jnp.pad(dec['b2'], (0, 6)).reshape(1, 61)
    p2p = jnp.pad(pos_dec['w2'], ((0, 0), (55, 0)))
    pb2p = jnp.pad(pos_dec['b2'], (55, 0)).reshape(1, 61)
    args = [q, z, cb6, otype, opos,
            dec['w0'], dec['b0'].reshape(1, -1),
            dec['w1'], dec['b1'].reshape(1, -1), d2p, db2p,
            pos_dec['w0'], pos_dec['b0'].reshape(1, -1),
            pos_dec['w1'], pos_dec['b1'].reshape(1, -1), p2p, pb2p]
    out, loss = pl.pallas_call(
        _final_kernel,
        in_specs=[pl.BlockSpec(a.shape, functools.partial(lambda n: (0,) * n, a.ndim))
                  for a in args],
        out_specs=[pl.BlockSpec((B, 61), lambda: (0, 0)),
                   pl.BlockSpec((1, 1), lambda: (0, 0))],
        out_shape=[jax.ShapeDtypeStruct((B, 61), jnp.float32),
                   jax.ShapeDtypeStruct((1, 1), jnp.float32)],
    )(*args)
    return out, loss


def kernel(obj_pc, hand_xyz, params):
    feat = _hand_pointnets(hand_xyz, params['hand_enc'])
    z, idxF = _emb_vq(feat, params['emb'], params['cb'])
    cbF = jnp.concatenate(params['cb'], axis=0)
    idxFp = jnp.pad(idxF.reshape(192).reshape(32, 6), ((0, 0), (0, 2)))
    qFp = _sc_vq(idxFp.reshape(256), cbF)
    otype, opos = _obj_pointnets(obj_pc, params['obj_type'],
                                 params['obj_pos'])
    out, loss = _final(qFp, z, params['cb6'], otype, opos,
                       params['dec'], params['pos_dec'])
    return out, loss[0, 0]
